# baseline (speedup 1.0000x reference)
import functools

import jax
import jax.numpy as jnp
from jax import lax
from jax.experimental import pallas as pl
from jax.experimental.pallas import tpu as pltpu
from jax.experimental.pallas import tpu_sc as plsc

BATCH = 16384
FIELDS = 26
DIM = 32
NUM_EMB = 1000000
NUM_ROWS = BATCH * FIELDS
NC = 2
NS = 16
NW = NC * NS
ROWS_PER_W = NUM_ROWS // NW
SUP = 512
NSUP = ROWS_PER_W // SUP

REG = 262144
BC = 8192
NBLK = REG // BC
LASTBLK = (NUM_EMB - 1) // BC
PACKED_ROWS = 4 * REG


def _pack_body(i0, i1, i2, i3, out_ref):
    s = jnp.concatenate([i0[...], i1[...], i2[...], i3[...]], axis=0)
    out_ref[...] = s.T


def _pack_in_spec(s):
    return pl.BlockSpec((DIM, BC), lambda j, s=s: (0, jnp.minimum(s * NBLK + j, LASTBLK)))


def _pack_table(embd_t):
    return pl.pallas_call(
        _pack_body,
        grid=(NBLK,),
        in_specs=[_pack_in_spec(s) for s in range(4)],
        out_specs=pl.BlockSpec((BC, 128), lambda j: (j, 0)),
        out_shape=jax.ShapeDtypeStruct((REG, 128), jnp.float32),
    )(embd_t, embd_t, embd_t, embd_t)


_mesh = plsc.VectorSubcoreMesh(core_axis_name="c", subcore_axis_name="s")


OUT_WORDS = NUM_ROWS * DIM
TW = 4096


@functools.partial(
    pl.kernel,
    mesh=_mesh,
    compiler_params=pltpu.CompilerParams(use_tc_tiling_on_sc=False,
                                         needs_layout_passes=False),
    out_type=jax.ShapeDtypeStruct((OUT_WORDS,), jnp.float32),
    scratch_types=[
        pltpu.VMEM((ROWS_PER_W,), jnp.int32),
        pltpu.VMEM((SUP, DIM), jnp.float32),
        pltpu.VMEM((SUP, DIM), jnp.float32),
        pltpu.VMEM((4 * TW,), jnp.float32),
        pltpu.VMEM((4 * TW,), jnp.float32),
        pltpu.SemaphoreType.DMA,
        pltpu.SemaphoreType.DMA,
        pltpu.SemaphoreType.DMA,
        pltpu.SemaphoreType.DMA,
    ],
)
def _gather_kernel(idx_hbm, table_hbm, out_hbm, idx_v, buf_a, buf_b,
                   tbuf_a, tbuf_b, ga, gb, wa, wb):
    wid = lax.axis_index("s") * NC + lax.axis_index("c")
    base = wid * ROWS_PER_W
    pltpu.sync_copy(idx_hbm.at[wid], idx_v)
    iota = lax.iota(jnp.int32, 16)

    def fire_gathers(s, buf, sem):
        pltpu.async_copy(table_hbm.at[idx_v.at[pl.ds(s * SUP, SUP)]],
                         buf, sem)

    def drain_gathers(s, buf, sem):
        pltpu.make_async_copy(table_hbm.at[idx_v.at[pl.ds(s * SUP, SUP)]],
                              buf, sem).wait()

    def out_off(s, t):
        p0 = base + s * SUP
        f = p0 >> 14
        blk0 = (p0 & 16383) >> 7
        return ((f * 4 + t) * 128 + blk0) * 1024

    def transpose_chunk(buf, tbuf):
        def tq(q, carry):
            t = q >> 2
            blk = q & 3
            rows = [iota + (blk * 128 + g * 16) for g in range(8)]
            dst0 = t * 4096 + blk * 1024
            for r in range(8):
                cols = jnp.broadcast_to(8 * t + r, (16,))
                for g in range(8):
                    v = plsc.load_gather(buf, [rows[g], cols])
                    tbuf[pl.ds(dst0 + r * 128 + g * 16, 16)] = v
            return carry

        lax.fori_loop(0, 16, tq, 0)

    def fire_write(s, tbuf, sem):
        for t in range(4):
            pltpu.async_copy(tbuf.at[pl.ds(t * TW, TW)],
                             out_hbm.at[pl.ds(out_off(s, t), TW)], sem)

    def drain_write(s, tbuf, sem):
        for t in range(4):
            pltpu.make_async_copy(tbuf.at[pl.ds(t * TW, TW)],
                                  out_hbm.at[pl.ds(out_off(s, t), TW)],
                                  sem).wait()

    fire_gathers(0, buf_a, ga)

    def body(i, carry):
        s0 = 2 * i
        drain_gathers(s0, buf_a, ga)
        fire_gathers(s0 + 1, buf_b, gb)
        transpose_chunk(buf_a, tbuf_a)

        @pl.when(i > 0)
        def _():
            drain_write(s0 - 1, tbuf_b, wb)

        fire_write(s0, tbuf_a, wa)
        drain_gathers(s0 + 1, buf_b, gb)

        @pl.when(i < NSUP // 2 - 1)
        def _():
            fire_gathers(s0 + 2, buf_a, ga)

        transpose_chunk(buf_b, tbuf_b)
        drain_write(s0, tbuf_a, wa)
        fire_write(s0 + 1, tbuf_b, wb)
        return carry

    lax.fori_loop(0, NSUP // 2, body, 0)
    drain_write(NSUP - 1, tbuf_b, wb)


def kernel(x, embd):
    table2 = _pack_table(embd.T)
    table = table2.reshape(PACKED_ROWS, DIM)
    remapped = ((x & (REG - 1)) << 2) | (x >> 18)
    idx = remapped.T.reshape(NW, ROWS_PER_W)
    out1d = _gather_kernel(idx, table)
    a = out1d.reshape(FIELDS, 4, 128, 8, 128)
    return a.transpose(2, 4, 0, 1, 3).reshape(BATCH, FIELDS, DIM)


# speedup vs baseline: 1.1683x; 1.1683x over previous
import functools

import jax
import jax.numpy as jnp
from jax import lax
from jax.experimental import pallas as pl
from jax.experimental.pallas import tpu as pltpu
from jax.experimental.pallas import tpu_sc as plsc

BATCH = 16384
FIELDS = 26
DIM = 32
NUM_EMB = 1000000
NUM_ROWS = BATCH * FIELDS
NC = 2
NS = 16
NW = NC * NS
ROWS_PER_W = NUM_ROWS // NW
SUP = 512
NSUP = ROWS_PER_W // SUP

REG = 262144
BC = 8192
NBLK = REG // BC
LASTBLK = (NUM_EMB - 1) // BC
PACKED_ROWS = 4 * REG


def _pack_body(i0, i1, i2, i3, out_ref):
    s = jnp.concatenate([i0[...], i1[...], i2[...], i3[...]], axis=0)
    out_ref[...] = s.T


def _pack_in_spec(s):
    return pl.BlockSpec((DIM, BC), lambda j, s=s: (0, jnp.minimum(s * NBLK + j, LASTBLK)))


def _pack_table(embd_t):
    return pl.pallas_call(
        _pack_body,
        grid=(NBLK,),
        in_specs=[_pack_in_spec(s) for s in range(4)],
        out_specs=pl.BlockSpec((BC, 128), lambda j: (j, 0)),
        out_shape=jax.ShapeDtypeStruct((REG, 128), jnp.float32),
    )(embd_t, embd_t, embd_t, embd_t)


_mesh = plsc.VectorSubcoreMesh(core_axis_name="c", subcore_axis_name="s")


OUT_WORDS = NUM_ROWS * DIM
TW = 4096


@functools.partial(
    pl.kernel,
    mesh=_mesh,
    compiler_params=pltpu.CompilerParams(use_tc_tiling_on_sc=False,
                                         needs_layout_passes=False),
    out_type=jax.ShapeDtypeStruct((OUT_WORDS,), jnp.float32),
    scratch_types=[
        pltpu.VMEM((ROWS_PER_W,), jnp.int32),
        pltpu.VMEM((SUP, DIM), jnp.float32),
        pltpu.VMEM((SUP, DIM), jnp.float32),
        pltpu.VMEM((4 * TW,), jnp.float32),
        pltpu.VMEM((4 * TW,), jnp.float32),
        pltpu.SemaphoreType.DMA,
        pltpu.SemaphoreType.DMA,
        pltpu.SemaphoreType.DMA,
        pltpu.SemaphoreType.DMA,
    ],
)
def _gather_kernel(idx_hbm, table_hbm, out_hbm, idx_v, buf_a, buf_b,
                   tbuf_a, tbuf_b, ga, gb, wa, wb):
    wid = lax.axis_index("s") * NC + lax.axis_index("c")
    base = wid * ROWS_PER_W
    pltpu.sync_copy(idx_hbm.at[wid], idx_v)
    iota = lax.iota(jnp.int32, 16)

    def fire_gathers(s, buf, sem):
        pltpu.async_copy(table_hbm.at[idx_v.at[pl.ds(s * SUP, SUP)]],
                         buf, sem)

    def drain_gathers(s, buf, sem):
        pltpu.make_async_copy(table_hbm.at[idx_v.at[pl.ds(s * SUP, SUP)]],
                              buf, sem).wait()

    def out_off(s, t):
        p0 = base + s * SUP
        f = p0 >> 14
        blk0 = (p0 & 16383) >> 7
        return ((f * 4 + t) * 128 + blk0) * 1024

    perm0 = (iota >> 3) * 4096 + (iota & 7) * 128
    perm1 = perm0 + 8192

    def transpose_chunk(buf, tbuf):
        def tq(q, carry):
            for u in range(4):
                p = q * 4 + u
                off = (p >> 7) * 1024 + (p & 127)
                v0 = buf[p, pl.ds(0, 16)]
                plsc.store_scatter(tbuf, [perm0 + off], v0)
                v1 = buf[p, pl.ds(16, 16)]
                plsc.store_scatter(tbuf, [perm1 + off], v1)
            return carry

        lax.fori_loop(0, SUP // 4, tq, 0)

    def fire_write(s, tbuf, sem):
        for t in range(4):
            pltpu.async_copy(tbuf.at[pl.ds(t * TW, TW)],
                             out_hbm.at[pl.ds(out_off(s, t), TW)], sem)

    def drain_write(s, tbuf, sem):
        for t in range(4):
            pltpu.make_async_copy(tbuf.at[pl.ds(t * TW, TW)],
                                  out_hbm.at[pl.ds(out_off(s, t), TW)],
                                  sem).wait()

    fire_gathers(0, buf_a, ga)

    def body(i, carry):
        s0 = 2 * i
        drain_gathers(s0, buf_a, ga)
        fire_gathers(s0 + 1, buf_b, gb)
        transpose_chunk(buf_a, tbuf_a)

        @pl.when(i > 0)
        def _():
            drain_write(s0 - 1, tbuf_b, wb)

        fire_write(s0, tbuf_a, wa)
        drain_gathers(s0 + 1, buf_b, gb)

        @pl.when(i < NSUP // 2 - 1)
        def _():
            fire_gathers(s0 + 2, buf_a, ga)

        transpose_chunk(buf_b, tbuf_b)
        drain_write(s0, tbuf_a, wa)
        fire_write(s0 + 1, tbuf_b, wb)
        return carry

    lax.fori_loop(0, NSUP // 2, body, 0)
    drain_write(NSUP - 1, tbuf_b, wb)


def kernel(x, embd):
    table2 = _pack_table(embd.T)
    table = table2.reshape(PACKED_ROWS, DIM)
    remapped = ((x & (REG - 1)) << 2) | (x >> 18)
    idx = remapped.T.reshape(NW, ROWS_PER_W)
    out1d = _gather_kernel(idx, table)
    a = out1d.reshape(FIELDS, 4, 128, 8, 128)
    return a.transpose(2, 4, 0, 1, 3).reshape(BATCH, FIELDS, DIM)
